# baseline (speedup 1.0000x reference)
import functools

import jax
import jax.numpy as jnp
from jax import lax
from jax.experimental import pallas as pl
from jax.experimental.pallas import tpu as pltpu
from jax.experimental.pallas import tpu_sc as plsc

EMB = 32
LANES = 16
IDXW = 128
QW = 512
KD = QW // IDXW
B = 4096
NQ = B // QW
WW = 800


@functools.lru_cache(maxsize=None)
def _make_gather(n_planes: int, vocab: int):
    info = plsc.get_sparse_core_info()
    nw = info.num_cores * info.num_subcores
    n_c = (n_planes + nw - 1) // nw
    rows_per_plane = B // IDXW
    n_win = vocab // WW
    n_wc = (n_win + nw - 1) // nw

    mesh = plsc.VectorSubcoreMesh(core_axis_name="c", subcore_axis_name="s")

    @functools.partial(
        pl.kernel,
        mesh=mesh,
        out_type=(
            jax.ShapeDtypeStruct((n_planes, EMB, B), jnp.float32),
            jax.ShapeDtypeStruct((vocab, EMB), jnp.float32),
        ),
        scratch_types=[
            pltpu.VMEM((rows_per_plane, IDXW), jnp.int32),
            pltpu.VMEM((2, QW, EMB), jnp.float32),
            pltpu.VMEM((2, EMB, QW + 1), jnp.float32),
            pltpu.VMEM((EMB, WW), jnp.float32),
            pltpu.VMEM((WW, EMB + 1), jnp.float32),
            pltpu.SemaphoreType.DMA((2,)),
            pltpu.SemaphoreType.DMA((2,)),
            pltpu.SemaphoreType.REGULAR,
        ],
        compiler_params=pltpu.CompilerParams(
            use_tc_tiling_on_sc=False, needs_layout_passes=False),
    )
    def gather(idx_hbm, tblt_hbm, out_hbm, trm_hbm,
               idx_v, gbuf, obuf, tin, tout, gsem, osem, bar_sem):
        cid = lax.axis_index("c")
        wid = lax.axis_index("s") * info.num_cores + cid
        iota = lax.iota(jnp.int32, LANES)

        def win(k, carry):
            j = k * nw + wid

            @pl.when(j < n_win)
            def _():
                v0 = j * WW
                pltpu.sync_copy(tblt_hbm.at[:, pl.ds(v0, WW)], tin)

                @plsc.parallel_loop(0, WW // LANES, unroll=4)
                def _(wg):
                    w_idx = iota + wg * LANES
                    for e in range(EMB):
                        val = tin[e, pl.ds(wg * LANES, LANES)]
                        e_vec = jnp.full((LANES,), e, jnp.int32)
                        plsc.store_scatter(tout, [w_idx, e_vec], val)

                pltpu.sync_copy(tout.at[:, pl.ds(0, EMB)],
                                trm_hbm.at[pl.ds(v0, WW)])

            return carry

        lax.fori_loop(0, n_wc, win, 0)

        plsc.subcore_barrier()
        pl.semaphore_signal(bar_sem, 1, core_index=1 - cid)
        pl.semaphore_wait(bar_sem, 1)

        def gcp(s, j, r):
            return pltpu.make_async_copy(
                trm_hbm.at[idx_v.at[r]],
                gbuf.at[s, pl.ds(j * IDXW, IDXW)],
                gsem.at[s])

        def ocp(t, q, s):
            return pltpu.make_async_copy(
                obuf.at[s, :, pl.ds(0, QW)],
                out_hbm.at[t, :, pl.ds(q * QW, QW)],
                osem.at[s])

        e_lo = iota
        e_hi = iota + LANES

        def plane(c, carry):
            t = c * nw + wid

            @pl.when(t < n_planes)
            def _():
                pltpu.sync_copy(
                    idx_hbm.at[pl.ds(t * rows_per_plane, rows_per_plane)],
                    idx_v)
                for j in range(KD):
                    gcp(0, j, j).start()
                for q in range(NQ):
                    s = q % 2
                    if q + 1 < NQ:
                        for j in range(KD):
                            gcp(1 - s, j, (q + 1) * KD + j).start()
                    for j in range(KD):
                        gcp(s, j, q * KD + j).wait()
                    if q >= 2:
                        ocp(t, q, s).wait()
                    else:
                        @pl.when(c > 0)
                        def _():
                            ocp(t, q, s).wait()

                    @plsc.parallel_loop(0, QW, unroll=8)
                    def _(w):
                        w_vec = jnp.full((LANES,), 0, jnp.int32) + w
                        lo = gbuf[s, w, pl.ds(0, LANES)]
                        hi = gbuf[s, w, pl.ds(LANES, LANES)]
                        plsc.store_scatter(obuf.at[s], [e_lo, w_vec], lo)
                        plsc.store_scatter(obuf.at[s], [e_hi, w_vec], hi)

                    ocp(t, q, s).start()

            return carry

        lax.fori_loop(0, n_c, plane, 0)
        ocp(0, 0, 0).wait()
        ocp(0, 0, 1).wait()

    return gather


def kernel(x, table):
    b, h = x.shape
    idx = jnp.asarray(x, jnp.int32).T.reshape(h * b // IDXW, IDXW)
    out, _ = _make_gather(h, table.shape[0])(idx, table.T)
    return jnp.transpose(out, (2, 0, 1))


# speedup vs baseline: 4.3324x; 4.3324x over previous
import functools

import jax
import jax.numpy as jnp
from jax import lax
from jax.experimental import pallas as pl
from jax.experimental.pallas import tpu as pltpu
from jax.experimental.pallas import tpu_sc as plsc

EMB = 32
LANES = 16
IDXW = 128
QW = 512
KD = QW // IDXW
B = 4096
NQ = B // QW


@functools.lru_cache(maxsize=None)
def _make_gather(n_planes: int, vocab: int):
    info = plsc.get_sparse_core_info()
    nw = info.num_cores * info.num_subcores
    n_c = (n_planes + nw - 1) // nw
    rows_per_plane = B // IDXW

    mesh = plsc.VectorSubcoreMesh(core_axis_name="c", subcore_axis_name="s")

    @functools.partial(
        pl.kernel,
        mesh=mesh,
        out_type=jax.ShapeDtypeStruct((n_planes, EMB, B), jnp.float32),
        scratch_types=[
            pltpu.VMEM((rows_per_plane, IDXW), jnp.int32),
            pltpu.VMEM((2, QW, EMB), jnp.float32),
            pltpu.VMEM((2, EMB, QW + 1), jnp.float32),
            pltpu.SemaphoreType.DMA((2,)),
            pltpu.SemaphoreType.DMA((2,)),
        ],
        compiler_params=pltpu.CompilerParams(
            use_tc_tiling_on_sc=False, needs_layout_passes=False),
    )
    def gather(idx_hbm, tbl_hbm, out_hbm, idx_v, gbuf, obuf, gsem, osem):
        wid = lax.axis_index("s") * info.num_cores + lax.axis_index("c")
        iota = lax.iota(jnp.int32, LANES)

        def gcp(s, j, r):
            return pltpu.make_async_copy(
                tbl_hbm.at[idx_v.at[r]],
                gbuf.at[s, pl.ds(j * IDXW, IDXW)],
                gsem.at[s])

        def ocp(t, q, s):
            return pltpu.make_async_copy(
                obuf.at[s, :, pl.ds(0, QW)],
                out_hbm.at[t, :, pl.ds(q * QW, QW)],
                osem.at[s])

        def plane(c, carry):
            t = c * nw + wid

            @pl.when(t < n_planes)
            def _():
                pltpu.sync_copy(
                    idx_hbm.at[pl.ds(t * rows_per_plane, rows_per_plane)],
                    idx_v)
                for j in range(KD):
                    gcp(0, j, j).start()
                for q in range(NQ):
                    s = q % 2
                    if q + 1 < NQ:
                        for j in range(KD):
                            gcp(1 - s, j, (q + 1) * KD + j).start()
                    for j in range(KD):
                        gcp(s, j, q * KD + j).wait()
                    if q >= 2:
                        ocp(t, q, s).wait()
                    else:
                        @pl.when(c > 0)
                        def _():
                            ocp(t, q, s).wait()

                    e_lo = iota
                    e_hi = iota + LANES

                    @plsc.parallel_loop(0, QW, unroll=16)
                    def _(w):
                        w_vec = jnp.full((LANES,), 0, jnp.int32) + w
                        lo = gbuf[s, w, pl.ds(0, LANES)]
                        hi = gbuf[s, w, pl.ds(LANES, LANES)]
                        plsc.store_scatter(obuf.at[s], [e_lo, w_vec], lo)
                        plsc.store_scatter(obuf.at[s], [e_hi, w_vec], hi)
                    ocp(t, q, s).start()

            return carry

        lax.fori_loop(0, n_c, plane, 0)
        ocp(0, 0, 0).wait()
        ocp(0, 0, 1).wait()

    return gather


def kernel(x, table):
    b, h = x.shape
    idx = jnp.asarray(x, jnp.int32).T.reshape(h * b // IDXW, IDXW)
    out = _make_gather(h, table.shape[0])(idx, table)
    return jnp.transpose(out, (2, 0, 1))


# speedup vs baseline: 4.3351x; 1.0006x over previous
import functools

import jax
import jax.numpy as jnp
from jax import lax
from jax.experimental import pallas as pl
from jax.experimental.pallas import tpu as pltpu
from jax.experimental.pallas import tpu_sc as plsc

EMB = 32
LANES = 16
IDXW = 128
QW = 512
KD = QW // IDXW
B = 4096
NQ = B // QW


@functools.lru_cache(maxsize=None)
def _make_gather(n_planes: int, vocab: int):
    info = plsc.get_sparse_core_info()
    nw = info.num_cores * info.num_subcores
    n_c = (n_planes + nw - 1) // nw
    rows_per_plane = B // IDXW

    mesh = plsc.VectorSubcoreMesh(core_axis_name="c", subcore_axis_name="s")

    @functools.partial(
        pl.kernel,
        mesh=mesh,
        out_type=jax.ShapeDtypeStruct((n_planes, EMB, B), jnp.float32),
        scratch_types=[
            pltpu.VMEM((rows_per_plane, IDXW), jnp.int32),
            pltpu.VMEM((2, QW, EMB), jnp.float32),
            pltpu.VMEM((2, EMB, QW + 1), jnp.float32),
            pltpu.SemaphoreType.DMA((2,)),
            pltpu.SemaphoreType.DMA((2,)),
        ],
        compiler_params=pltpu.CompilerParams(
            use_tc_tiling_on_sc=False, needs_layout_passes=False),
    )
    def gather(idx_hbm, tbl_hbm, out_hbm, idx_v, gbuf, obuf, gsem, osem):
        wid = lax.axis_index("s") * info.num_cores + lax.axis_index("c")
        iota = lax.iota(jnp.int32, LANES)

        def gcp(s, j, r):
            return pltpu.make_async_copy(
                tbl_hbm.at[idx_v.at[r]],
                gbuf.at[s, pl.ds(j * IDXW, IDXW)],
                gsem.at[s])

        def ocp(t, q, s):
            return pltpu.make_async_copy(
                obuf.at[s, :, pl.ds(0, QW)],
                out_hbm.at[t, :, pl.ds(q * QW, QW)],
                osem.at[s])

        def plane(c, carry):
            t = c * nw + wid

            @pl.when(t < n_planes)
            def _():
                pltpu.sync_copy(
                    idx_hbm.at[pl.ds(t * rows_per_plane, rows_per_plane)],
                    idx_v)
                for j in range(KD):
                    gcp(0, j, j).start()
                for q in range(NQ):
                    s = q % 2
                    if q + 1 < NQ:
                        for j in range(KD):
                            gcp(1 - s, j, (q + 1) * KD + j).start()
                    for j in range(KD):
                        gcp(s, j, q * KD + j).wait()
                    if q >= 2:
                        ocp(t, q, s).wait()
                    else:
                        @pl.when(c > 0)
                        def _():
                            ocp(t, q, s).wait()

                    e_lo = iota
                    e_hi = iota + LANES

                    @plsc.parallel_loop(0, QW, unroll=8)
                    def _(w):
                        w_vec = jnp.full((LANES,), 0, jnp.int32) + w
                        lo = gbuf[s, w, pl.ds(0, LANES)]
                        hi = gbuf[s, w, pl.ds(LANES, LANES)]
                        plsc.store_scatter(obuf.at[s], [e_lo, w_vec], lo)
                        plsc.store_scatter(obuf.at[s], [e_hi, w_vec], hi)
                    ocp(t, q, s).start()

            return carry

        lax.fori_loop(0, n_c, plane, 0)
        ocp(0, 0, 0).wait()
        ocp(0, 0, 1).wait()

    return gather


def kernel(x, table):
    b, h = x.shape
    idx = jnp.asarray(x, jnp.int32).T.reshape(h * b // IDXW, IDXW)
    out = _make_gather(h, table.shape[0])(idx, table)
    return jnp.transpose(out, (2, 0, 1))
